# baseline (speedup 1.0000x reference)
import jax
import jax.numpy as jnp
from jax import lax
from jax.experimental import pallas as pl
from jax.experimental.pallas import tpu as pltpu
from jax.experimental.pallas import tpu_sc as plsc

EMBEDDING_DIM = 32
NUM_CORES = 2
NUM_SUBCORES = 16
NUM_WORKERS = NUM_CORES * NUM_SUBCORES

V = 1_000_000
D = EMBEDDING_DIM
VBLK = 128
NFULL = V // VBLK
TAIL = V - NFULL * VBLK
PACKROWS = V * D // 128

CHUNK = 1600
NBUF = 2


def _wid():
    return lax.axis_index("s") * NUM_CORES + lax.axis_index("c")


def _wpack_body(wt_hbm, wpack_hbm, band_v, tr_v, band_t, tr_t,
                sem_i0, sem_i1, sem_o0, sem_o1):
    wid = _wid()
    nblk = (NFULL - wid + NUM_WORKERS - 1) // NUM_WORKERS
    iota16 = lax.iota(jnp.int32, 16)
    base32 = [(iota16 + j0 * 16) * D for j0 in range(8)]
    sem_i = (sem_i0, sem_i1)
    sem_o = (sem_o0, sem_o1)

    def start_in(vb, bb):
        pltpu.make_async_copy(
            wt_hbm.at[:, pl.ds(vb * VBLK, VBLK)], band_v.at[bb], sem_i[bb]
        ).start()

    def wait_in(bb):
        pltpu.make_async_copy(
            wt_hbm.at[:, pl.ds(0, VBLK)], band_v.at[bb], sem_i[bb]
        ).wait()

    def start_out(vb, bb):
        pltpu.make_async_copy(
            tr_v.at[bb], wpack_hbm.at[pl.ds(vb * (VBLK * D // 128), VBLK * D // 128)],
            sem_o[bb],
        ).start()

    def wait_out(bb):
        pltpu.make_async_copy(
            tr_v.at[bb], wpack_hbm.at[pl.ds(0, VBLK * D // 128)], sem_o[bb]
        ).wait()

    def transpose_block(bb):
        def dbody(d, carry):
            for j0 in range(8):
                vec = band_v[bb, d, pl.ds(j0 * 16, 16)]
                flat = base32[j0] + d
                plsc.store_scatter(
                    tr_v.at[bb],
                    [lax.shift_right_logical(flat, 7),
                     lax.bitwise_and(flat, 127)],
                    vec)
            return carry
        lax.fori_loop(0, D, dbody, 0, unroll=2)

    start_in(wid, 0)
    start_in(wid + NUM_WORKERS, 1)
    for bb in range(2):
        wait_in(bb)
        transpose_block(bb)
        start_out(wid + bb * NUM_WORKERS, bb)
        start_in(wid + (bb + 2) * NUM_WORKERS, bb)

    def group(g, carry):
        for bb in range(2):
            k = 2 + g * 2 + bb
            vb = wid + k * NUM_WORKERS

            @pl.when(k < nblk)
            def _():
                wait_in(bb)
                wait_out(bb)
                transpose_block(bb)
                start_out(vb, bb)

            @pl.when(k + 2 < nblk)
            def _():
                pltpu.make_async_copy(
                    wt_hbm.at[:, pl.ds((vb + 2 * NUM_WORKERS) * VBLK, VBLK)],
                    band_v.at[bb], sem_i[bb],
                ).start()
        return carry

    ngroups = (245 - 2 + 1) // 2
    lax.fori_loop(0, ngroups, group, 0, unroll=False)
    for bb in range(2):
        wait_out(bb)

    @pl.when(wid == NUM_WORKERS - 1)
    def _():
        pltpu.sync_copy(wt_hbm.at[:, pl.ds(NFULL * VBLK, TAIL)], band_t)

        def dbody(d, carry):
            for j0 in range(TAIL // 16):
                vec = band_t[d, pl.ds(j0 * 16, 16)]
                flat = base32[j0] + d
                plsc.store_scatter(
                    tr_t,
                    [lax.shift_right_logical(flat, 7),
                     lax.bitwise_and(flat, 127)],
                    vec)
            return carry
        lax.fori_loop(0, D, dbody, 0, unroll=2)
        pltpu.sync_copy(
            tr_t, wpack_hbm.at[pl.ds(NFULL * (VBLK * D // 128), TAIL * D // 128)])


def _emb_body(idx_hbm, table_hbm, out_hbm,
              idx_v, rows_v, sem_i0, sem_i1, sem_g0, sem_g1, sem_s0, sem_s1):
    wid = _wid()
    n_per_w = idx_hbm.shape[0] // NUM_WORKERS
    nchunks = n_per_w // CHUNK
    base = wid * n_per_w

    sem_i = (sem_i0, sem_i1)
    sem_g = (sem_g0, sem_g1)
    sem_s = (sem_s0, sem_s1)

    def start_idx(g, b):
        pltpu.make_async_copy(
            idx_hbm.at[pl.ds(base + g * CHUNK, CHUNK)], idx_v.at[b], sem_i[b]
        ).start()

    def wait_idx(b):
        pltpu.make_async_copy(
            idx_hbm.at[pl.ds(base, CHUNK)], idx_v.at[b], sem_i[b]
        ).wait()

    def start_gather(b):
        pltpu.make_async_copy(
            table_hbm.at[idx_v.at[b]], rows_v.at[b], sem_g[b]
        ).start()

    def wait_gather(b):
        pltpu.make_async_copy(
            table_hbm.at[idx_v.at[b]], rows_v.at[b], sem_g[b]
        ).wait()

    def start_store(g, b):
        pltpu.make_async_copy(
            rows_v.at[b], out_hbm.at[pl.ds(base + g * CHUNK, CHUNK)], sem_s[b]
        ).start()

    def wait_store(b):
        pltpu.make_async_copy(
            rows_v.at[b], out_hbm.at[pl.ds(base, CHUNK)], sem_s[b]
        ).wait()

    start_idx(0, 0)
    start_idx(1, 1)
    for b in range(NBUF):
        wait_idx(b)
        start_gather(b)
        wait_gather(b)
        start_store(b, b)
        start_idx(b + NBUF, b)

    def group_body(gr, carry):
        for b in range(NBUF):
            g = NBUF + gr * NBUF + b
            wait_idx(b)
            wait_store(b)
            start_gather(b)
            wait_gather(b)
            start_store(g, b)
            start_idx(g + NBUF, b)
        return carry

    ngroups = (nchunks - 2 * NBUF) // NBUF
    lax.fori_loop(0, ngroups, group_body, 0, unroll=False)

    for b in range(NBUF):
        g = nchunks - NBUF + b
        wait_idx(b)
        wait_store(b)
        start_gather(b)
        wait_gather(b)
        start_store(g, b)
    for b in range(NBUF):
        wait_store(b)


T = 200
B = 16384


def _relayout_body(g2_hbm, out_hbm, src_v, tile_v,
                   sem_i0, sem_i1, sem_o0, sem_o1):
    wid = _wid()
    iota16 = lax.iota(jnp.int32, 16)
    base32 = [(iota16 + j0 * 16) * D for j0 in range(8)]
    sem_i = (sem_i0, sem_i1)
    sem_o = (sem_o0, sem_o1)
    NUunits = (T * (B // 128)) // NUM_WORKERS
    u0 = wid * NUunits

    def src_row(u):
        return (u // 128) * (B * D // 128) + lax.rem(u, 128) * (128 * D // 128)

    def start_in(u, bb):
        pltpu.make_async_copy(
            g2_hbm.at[pl.ds(src_row(u), 128 * D // 128)], src_v.at[bb], sem_i[bb]
        ).start()

    def wait_in(bb):
        pltpu.make_async_copy(
            g2_hbm.at[pl.ds(0, 128 * D // 128)], src_v.at[bb], sem_i[bb]
        ).wait()

    def start_out(u, bb):
        pltpu.make_async_copy(
            tile_v.at[bb],
            out_hbm.at[u // 128, :, pl.ds(lax.rem(u, 128) * 128, 128)],
            sem_o[bb],
        ).start()

    def wait_out(bb):
        pltpu.make_async_copy(
            tile_v.at[bb], out_hbm.at[0, :, pl.ds(0, 128)], sem_o[bb]
        ).wait()

    def transpose_unit(bb):
        def dbody(d, carry):
            for j0 in range(8):
                flat = base32[j0] + d
                vec = plsc.load_gather(
                    src_v.at[bb],
                    [lax.shift_right_logical(flat, 7),
                     lax.bitwise_and(flat, 127)])
                tile_v[bb, d, pl.ds(j0 * 16, 16)] = vec
            return carry
        lax.fori_loop(0, D, dbody, 0, unroll=2)

    start_in(u0, 0)
    start_in(u0 + 1, 1)
    for bb in range(2):
        wait_in(bb)
        transpose_unit(bb)
        start_out(u0 + bb, bb)
        start_in(u0 + bb + 2, bb)

    def pairs(p, carry):
        for bb in range(2):
            k = 2 + 2 * p + bb
            u = u0 + k
            wait_in(bb)
            wait_out(bb)
            transpose_unit(bb)
            start_out(u, bb)

            @pl.when(k + 2 < NUunits)
            def _():
                start_in(u + 2, bb)
        return carry

    lax.fori_loop(0, (NUunits - 2) // 2, pairs, 0, unroll=False)
    for bb in range(2):
        wait_out(bb)


def kernel(token_ids, W):
    Bsz, H = token_ids.shape
    flat = jnp.swapaxes(token_ids, 0, 1).reshape(-1).astype(jnp.int32)
    n = flat.shape[0]
    mesh = plsc.VectorSubcoreMesh(core_axis_name="c", subcore_axis_name="s")

    g2 = pl.kernel(
        _emb_body,
        mesh=mesh,
        compiler_params=pltpu.CompilerParams(use_tc_tiling_on_sc=False),
        out_type=jax.ShapeDtypeStruct((n, EMBEDDING_DIM), jnp.float32),
        scratch_types=[
            pltpu.VMEM((NBUF, CHUNK), jnp.int32),
            pltpu.VMEM((NBUF, CHUNK, EMBEDDING_DIM), jnp.float32),
            pltpu.SemaphoreType.DMA,
            pltpu.SemaphoreType.DMA,
            pltpu.SemaphoreType.DMA,
            pltpu.SemaphoreType.DMA,
            pltpu.SemaphoreType.DMA,
            pltpu.SemaphoreType.DMA,
        ],
    )(flat, W)

    g2p = g2.reshape(n * D // 128, 128)

    out = pl.kernel(
        _relayout_body,
        mesh=mesh,
        compiler_params=pltpu.CompilerParams(
            use_tc_tiling_on_sc=True, needs_layout_passes=False),
        out_type=jax.ShapeDtypeStruct((T, D, B), jnp.float32),
        scratch_types=[
            pltpu.VMEM((2, 128 * D // 128, 128), jnp.float32),
            pltpu.VMEM((2, D, 128), jnp.float32),
            pltpu.SemaphoreType.DMA,
            pltpu.SemaphoreType.DMA,
            pltpu.SemaphoreType.DMA,
            pltpu.SemaphoreType.DMA,
        ],
    )(g2p)

    return jnp.transpose(out, (2, 0, 1))
